# baseline (speedup 1.0000x reference)
import functools

import jax
import jax.numpy as jnp
from jax import lax
from jax.experimental import pallas as pl
from jax.experimental.pallas import tpu as pltpu
from jax.experimental.pallas import tpu_sc as plsc

B = 256
D = 128
M = 100000
K = 4096
T = 0.07
MOMENTUM = 0.5

NC, NS = 2, 16
NW = NC * NS
BPW = B // NW

_f32 = jnp.float32


def _norm_rows(x):
    n = jnp.sqrt(jnp.sum(x * x, axis=1, keepdims=True))
    return x / jnp.maximum(n, 1e-12)


_TE = 4096
_NBLK = pl.cdiv(M, _TE)
_WPAD = _NBLK * _TE


def _k1_body(q_ref, ql_ref, bank_ref, L_ref, qn_ref, qi_s):
    i32 = jnp.int32

    @pl.when(pl.program_id(0) == 0)
    def _():
        qg = _norm_rows(q_ref[...])
        qj = _norm_rows(ql_ref[...])
        qn_ref[0:B, :] = qg
        qn_ref[B:2 * B, :] = qj
        qs = jnp.concatenate([qg, qj], axis=0)
        ii = lax.broadcasted_iota(i32, (2 * B, 2 * B), 0)
        jj = lax.broadcasted_iota(i32, (2 * B, 2 * B), 1)
        tgt = jnp.right_shift(ii, 1) + lax.bitwise_and(ii, 1) * B
        perm = (jj == tgt).astype(_f32)
        qi_s[...] = lax.dot_general(perm, qs, (((1,), (0,)), ((), ())),
                                    preferred_element_type=_f32)

    bt = bank_ref[...].astype(jnp.bfloat16)
    li = lax.dot_general(qi_s[...].astype(jnp.bfloat16), bt,
                         (((1,), (1,)), ((), ())),
                         preferred_element_type=_f32)
    L_ref[...] = pltpu.bitcast(li.astype(jnp.bfloat16), jnp.int32)


def _k1(q, q_l, bank):
    return pl.pallas_call(
        _k1_body,
        grid=(_NBLK,),
        in_specs=[
            pl.BlockSpec((B, D), lambda i: (0, 0)),
            pl.BlockSpec((B, D), lambda i: (0, 0)),
            pl.BlockSpec((_TE, D), lambda i: (i, 0)),
        ],
        out_specs=[
            pl.BlockSpec((B, _TE), lambda i: (0, i)),
            pl.BlockSpec((2 * B, D), lambda i: (0, 0)),
        ],
        out_shape=[
            jax.ShapeDtypeStruct((B, _WPAD), jnp.int32),
            jax.ShapeDtypeStruct((2 * B, D), _f32),
        ],
        scratch_shapes=[pltpu.VMEM((2 * B, D), _f32)],
    )(q, q_l, bank)


def _k0_body(bank_hbm, idx_hbm, posf_hbm, pidx_v, prow_v, sem_p):
    wid = lax.axis_index("s") * NC + lax.axis_index("c")
    b0 = wid * BPW
    pltpu.sync_copy(idx_hbm.at[pl.ds(b0, BPW)], pidx_v)
    pltpu.async_copy(bank_hbm.at[pidx_v], prow_v, sem_p).wait()
    pltpu.sync_copy(prow_v, posf_hbm.at[pl.ds(b0, BPW)])


_k0 = pl.kernel(
    _k0_body,
    out_type=jax.ShapeDtypeStruct((B, D), _f32),
    mesh=plsc.VectorSubcoreMesh(core_axis_name="c", subcore_axis_name="s",
                                num_cores=NC, num_subcores=NS),
    scratch_types=[
        pltpu.VMEM((BPW,), jnp.int32),
        pltpu.VMEM((BPW, D), _f32),
        pltpu.SemaphoreType.DMA,
    ],
    compiler_params=pltpu.CompilerParams(needs_layout_passes=False),
)


def _k2_body(L_hbm, nidx_hbm, zsum_hbm, row_v, im_v, zg_v, zj_v, s_row):
    wid = lax.axis_index("s") * NC + lax.axis_index("c")
    b0 = wid * BPW
    r_inv_t = jnp.float32(1.0 / T)

    rowcopy = pltpu.async_copy(L_hbm.at[b0], row_v, s_row)
    for bi in range(BPW):
        pltpu.sync_copy(nidx_hbm.at[b0 + bi], im_v)
        rowcopy.wait()

        def u_body(i, acc):
            accg, accj = acc
            off = pl.multiple_of(i * 16, 16)
            wv = plsc.load_gather(row_v, [im_v[pl.ds(off, 16)]])
            lg = plsc.bitcast(jnp.left_shift(wv, 16), _f32)
            lj = plsc.bitcast(lax.bitwise_and(wv, jnp.int32(-65536)), _f32)
            accg = accg + jnp.exp(lg * r_inv_t)
            accj = accj + jnp.exp(lj * r_inv_t)
            return accg, accj

        z0 = jnp.zeros((16,), _f32)
        accg, accj = lax.fori_loop(0, K // 16, u_body, (z0, z0), unroll=4)
        if bi + 1 < BPW:
            rowcopy = pltpu.async_copy(L_hbm.at[b0 + bi + 1], row_v, s_row)
        zg_v[...] = accg
        zj_v[...] = accj
        pltpu.sync_copy(zg_v, zsum_hbm.at[0, b0 + bi])
        pltpu.sync_copy(zj_v, zsum_hbm.at[1, b0 + bi])


_k2 = pl.kernel(
    _k2_body,
    out_type=jax.ShapeDtypeStruct((2, B, 16), _f32),
    mesh=plsc.VectorSubcoreMesh(core_axis_name="c", subcore_axis_name="s",
                                num_cores=NC, num_subcores=NS),
    scratch_types=[
        pltpu.VMEM((_WPAD,), jnp.int32),
        pltpu.VMEM((K,), jnp.int32),
        pltpu.VMEM((16,), _f32),
        pltpu.VMEM((16,), _f32),
        pltpu.SemaphoreType.DMA,
    ],
    compiler_params=pltpu.CompilerParams(needs_layout_passes=False),
)


_RB = 1024


def _k4_body(qn_ref, posf_ref, idx_ref, bank_ref, out_ref, upd_s):
    i = pl.program_id(0)
    i32 = jnp.int32

    @pl.when(i == 0)
    def _():
        qn = qn_ref[...]
        pf = posf_ref[...]
        u = _norm_rows(MOMENTUM * pf + (1.0 - MOMENTUM) * qn[0:B, :])
        idxr = idx_ref[...].astype(_f32)
        eye = (lax.broadcasted_iota(i32, (B, B), 0)
               == lax.broadcasted_iota(i32, (B, B), 1)).astype(_f32)
        dnT = (((1,), (1,)), ((), ()))
        idxc = lax.dot_general(eye, idxr, dnT,
                               preferred_element_type=_f32)
        eqm = (jnp.broadcast_to(idxc, (B, B))
               == jnp.broadcast_to(idxr, (B, B))).astype(_f32)
        gt = (lax.broadcasted_iota(i32, (B, B), 0)
              > lax.broadcasted_iota(i32, (B, B), 1)).astype(_f32)
        dup = jnp.sum(eqm * gt, axis=0, keepdims=True)
        keep = (dup == 0.0).astype(_f32)
        keepc = lax.dot_general(eye, keep, dnT,
                                preferred_element_type=_f32)
        upd_s[...] = u * keepc

    rows = lax.broadcasted_iota(i32, (_RB, B), 0) + i * _RB
    idxb = jnp.broadcast_to(idx_ref[...], (_RB, B))
    eq = (rows == idxb).astype(_f32)
    hit = jnp.sum(eq, axis=1, keepdims=True)
    miss = (hit == 0.0).astype(_f32)
    upd = lax.dot_general(eq, upd_s[...], (((1,), (0,)), ((), ())),
                          preferred_element_type=_f32)
    out_ref[...] = miss * bank_ref[...] + upd


def _k4(qn, posf, idx_row, bank):
    grid = (pl.cdiv(M, _RB),)
    return pl.pallas_call(
        _k4_body,
        grid=grid,
        in_specs=[
            pl.BlockSpec((2 * B, D), lambda i: (0, 0)),
            pl.BlockSpec((B, D), lambda i: (0, 0)),
            pl.BlockSpec((1, B), lambda i: (0, 0)),
            pl.BlockSpec((_RB, D), lambda i: (i, 0)),
        ],
        out_specs=pl.BlockSpec((_RB, D), lambda i: (i, 0)),
        out_shape=jax.ShapeDtypeStruct((M, D), _f32),
        scratch_shapes=[pltpu.VMEM((B, D), _f32)],
    )(qn, posf, idx_row, bank)


def _k5_body(qn_ref, posf_ref, zsum_ref, loss_ref):
    qn = qn_ref[...]
    pf = posf_ref[...]
    pf2 = jnp.concatenate([pf, pf], axis=0)
    pos = jnp.sum(pf2 * qn, axis=1, keepdims=True)
    zs = zsum_ref[...].reshape(2 * B, 16)
    z = jnp.sum(zs, axis=1, keepdims=True) + jnp.exp(pos / T)
    lpb = jnp.log(z) - pos / T
    loss_ref[...] = jnp.mean(lpb, keepdims=True)


def _k5(qn, posf, zsum):
    return pl.pallas_call(
        _k5_body,
        in_specs=[
            pl.BlockSpec((2 * B, D), lambda: (0, 0)),
            pl.BlockSpec((B, D), lambda: (0, 0)),
            pl.BlockSpec((2, B, 16), lambda: (0, 0, 0)),
        ],
        out_specs=pl.BlockSpec((1, 1), lambda: (0, 0)),
        out_shape=jax.ShapeDtypeStruct((1, 1), _f32),
    )(qn, posf, zsum)


def kernel(q, q_l, feature_bank, idx, neg_idx):
    idx_row = idx.reshape(1, B)
    posf = _k0(feature_bank, idx)
    Lmat, qn = _k1(q, q_l, feature_bank)
    zsum = _k2(Lmat, neg_idx.reshape(B, K))
    new_bank = _k4(qn, posf, idx_row, feature_bank)
    loss = _k5(qn, posf, zsum)
    return loss[0, 0], new_bank


# speedup vs baseline: 1.2205x; 1.2205x over previous
import functools

import jax
import jax.numpy as jnp
from jax import lax
from jax.experimental import pallas as pl
from jax.experimental.pallas import tpu as pltpu
from jax.experimental.pallas import tpu_sc as plsc

B = 256
D = 128
M = 100000
K = 4096
T = 0.07
MOMENTUM = 0.5

NC, NS = 2, 16
NW = NC * NS
BPW = B // NW

_f32 = jnp.float32


def _norm_rows(x):
    n = jnp.sqrt(jnp.sum(x * x, axis=1, keepdims=True))
    return x / jnp.maximum(n, 1e-12)


_TE = 4096
_NBLK = pl.cdiv(M, _TE)
_WPAD = _NBLK * _TE


def _k1_body(q_ref, ql_ref, bank_ref, L_ref, qn_ref, qi_s):
    i32 = jnp.int32

    @pl.when(pl.program_id(0) == 0)
    def _():
        qg = _norm_rows(q_ref[...])
        qj = _norm_rows(ql_ref[...])
        qn_ref[0:B, :] = qg
        qn_ref[B:2 * B, :] = qj
        qs = jnp.concatenate([qg, qj], axis=0)
        ii = lax.broadcasted_iota(i32, (2 * B, 2 * B), 0)
        jj = lax.broadcasted_iota(i32, (2 * B, 2 * B), 1)
        tgt = jnp.right_shift(ii, 1) + lax.bitwise_and(ii, 1) * B
        perm = (jj == tgt).astype(_f32)
        qi_s[...] = lax.dot_general(perm, qs, (((1,), (0,)), ((), ())),
                                    preferred_element_type=_f32)

    bt = bank_ref[...].astype(jnp.bfloat16)
    li = lax.dot_general(qi_s[...].astype(jnp.bfloat16), bt,
                         (((1,), (1,)), ((), ())),
                         preferred_element_type=_f32)
    L_ref[...] = pltpu.bitcast(li.astype(jnp.bfloat16), jnp.int32)


def _k1(q, q_l, bank):
    return pl.pallas_call(
        _k1_body,
        grid=(_NBLK,),
        in_specs=[
            pl.BlockSpec((B, D), lambda i: (0, 0)),
            pl.BlockSpec((B, D), lambda i: (0, 0)),
            pl.BlockSpec((_TE, D), lambda i: (i, 0)),
        ],
        out_specs=[
            pl.BlockSpec((B, _TE), lambda i: (0, i)),
            pl.BlockSpec((2 * B, D), lambda i: (0, 0)),
        ],
        out_shape=[
            jax.ShapeDtypeStruct((B, _WPAD), jnp.int32),
            jax.ShapeDtypeStruct((2 * B, D), _f32),
        ],
        scratch_shapes=[pltpu.VMEM((2 * B, D), _f32)],
    )(q, q_l, bank)


def _k0_body(bank_hbm, idx_hbm, posf_hbm, pidx_v, prow_v, sem_p):
    wid = lax.axis_index("s") * NC + lax.axis_index("c")
    b0 = wid * BPW
    pltpu.sync_copy(idx_hbm.at[pl.ds(b0, BPW)], pidx_v)
    pltpu.async_copy(bank_hbm.at[pidx_v], prow_v, sem_p).wait()
    pltpu.sync_copy(prow_v, posf_hbm.at[pl.ds(b0, BPW)])


_k0 = pl.kernel(
    _k0_body,
    out_type=jax.ShapeDtypeStruct((B, D), _f32),
    mesh=plsc.VectorSubcoreMesh(core_axis_name="c", subcore_axis_name="s",
                                num_cores=NC, num_subcores=NS),
    scratch_types=[
        pltpu.VMEM((BPW,), jnp.int32),
        pltpu.VMEM((BPW, D), _f32),
        pltpu.SemaphoreType.DMA,
    ],
    compiler_params=pltpu.CompilerParams(needs_layout_passes=False),
)


def _k2_body(L_hbm, nidx_hbm, zsum_hbm, row_v, im_v, zg_v, zj_v, s_row):
    wid = lax.axis_index("s") * NC + lax.axis_index("c")
    b0 = wid * BPW
    r_inv_t = jnp.float32(1.0 / T)

    rowcopy = pltpu.async_copy(L_hbm.at[b0], row_v, s_row)
    for bi in range(BPW):
        pltpu.sync_copy(nidx_hbm.at[b0 + bi], im_v)
        rowcopy.wait()

        def u_body(i, acc):
            accg, accj = acc
            off = pl.multiple_of(i * 16, 16)
            wv = plsc.load_gather(row_v, [im_v[pl.ds(off, 16)]])
            lg = plsc.bitcast(jnp.left_shift(wv, 16), _f32)
            lj = plsc.bitcast(lax.bitwise_and(wv, jnp.int32(-65536)), _f32)
            accg = accg + jnp.exp(lg * r_inv_t)
            accj = accj + jnp.exp(lj * r_inv_t)
            return accg, accj

        z0 = jnp.zeros((16,), _f32)
        accg, accj = lax.fori_loop(0, K // 16, u_body, (z0, z0), unroll=4)
        if bi + 1 < BPW:
            rowcopy = pltpu.async_copy(L_hbm.at[b0 + bi + 1], row_v, s_row)
        zg_v[...] = accg
        zj_v[...] = accj
        pltpu.sync_copy(zg_v, zsum_hbm.at[0, b0 + bi])
        pltpu.sync_copy(zj_v, zsum_hbm.at[1, b0 + bi])


_k2 = pl.kernel(
    _k2_body,
    out_type=jax.ShapeDtypeStruct((2, B, 16), _f32),
    mesh=plsc.VectorSubcoreMesh(core_axis_name="c", subcore_axis_name="s",
                                num_cores=NC, num_subcores=NS),
    scratch_types=[
        pltpu.VMEM((_WPAD,), jnp.int32),
        pltpu.VMEM((K,), jnp.int32),
        pltpu.VMEM((16,), _f32),
        pltpu.VMEM((16,), _f32),
        pltpu.SemaphoreType.DMA,
    ],
    compiler_params=pltpu.CompilerParams(needs_layout_passes=False),
)


def _k3_body(qn_ref, posf_ref, idx_ref, upd_ref):
    i32 = jnp.int32
    qn = qn_ref[...]
    pf = posf_ref[...]
    u = _norm_rows(MOMENTUM * pf + (1.0 - MOMENTUM) * qn[0:B, :])
    idxr = idx_ref[...].astype(_f32)
    eye = (lax.broadcasted_iota(i32, (B, B), 0)
           == lax.broadcasted_iota(i32, (B, B), 1)).astype(_f32)
    dnT = (((1,), (1,)), ((), ()))
    idxc = lax.dot_general(eye, idxr, dnT,
                           preferred_element_type=_f32)
    eqm = (jnp.broadcast_to(idxc, (B, B))
           == jnp.broadcast_to(idxr, (B, B))).astype(_f32)
    gt = (lax.broadcasted_iota(i32, (B, B), 0)
          > lax.broadcasted_iota(i32, (B, B), 1)).astype(_f32)
    dup = jnp.sum(eqm * gt, axis=0, keepdims=True)
    keep = (dup == 0.0).astype(_f32)
    keepc = lax.dot_general(eye, keep, dnT, preferred_element_type=_f32)
    upd_ref[...] = u * keepc


def _k3(qn, posf, idx_row):
    return pl.pallas_call(
        _k3_body,
        in_specs=[
            pl.BlockSpec((2 * B, D), lambda: (0, 0)),
            pl.BlockSpec((B, D), lambda: (0, 0)),
            pl.BlockSpec((1, B), lambda: (0, 0)),
        ],
        out_specs=pl.BlockSpec((B, D), lambda: (0, 0)),
        out_shape=jax.ShapeDtypeStruct((B, D), _f32),
    )(qn, posf, idx_row)


_RB = 1024


def _k4_body(bank_ref, idx_ref, upd_ref, out_ref):
    i = pl.program_id(0)
    i32 = jnp.int32
    rows = lax.broadcasted_iota(i32, (_RB, B), 0) + i * _RB
    idxb = jnp.broadcast_to(idx_ref[...], (_RB, B))
    eq = (rows == idxb).astype(_f32)
    hit = jnp.sum(eq, axis=1, keepdims=True)
    miss = (hit == 0.0).astype(_f32)
    upd = lax.dot_general(eq, upd_ref[...], (((1,), (0,)), ((), ())),
                          preferred_element_type=_f32)
    out_ref[...] = miss * bank_ref[...] + upd


def _k4(bank, idx_row, upd):
    grid = (pl.cdiv(M, _RB),)
    return pl.pallas_call(
        _k4_body,
        grid=grid,
        in_specs=[
            pl.BlockSpec((_RB, D), lambda i: (i, 0)),
            pl.BlockSpec((1, B), lambda i: (0, 0)),
            pl.BlockSpec((B, D), lambda i: (0, 0)),
        ],
        out_specs=pl.BlockSpec((_RB, D), lambda i: (i, 0)),
        out_shape=jax.ShapeDtypeStruct((M, D), _f32),
    )(bank, idx_row, upd)


def _k5_body(qn_ref, posf_ref, zsum_ref, loss_ref):
    qn = qn_ref[...]
    pf = posf_ref[...]
    pf2 = jnp.concatenate([pf, pf], axis=0)
    pos = jnp.sum(pf2 * qn, axis=1, keepdims=True)
    zs = zsum_ref[...].reshape(2 * B, 16)
    z = jnp.sum(zs, axis=1, keepdims=True) + jnp.exp(pos / T)
    lpb = jnp.log(z) - pos / T
    loss_ref[...] = jnp.mean(lpb, keepdims=True)


def _k5(qn, posf, zsum):
    return pl.pallas_call(
        _k5_body,
        in_specs=[
            pl.BlockSpec((2 * B, D), lambda: (0, 0)),
            pl.BlockSpec((B, D), lambda: (0, 0)),
            pl.BlockSpec((2, B, 16), lambda: (0, 0, 0)),
        ],
        out_specs=pl.BlockSpec((1, 1), lambda: (0, 0)),
        out_shape=jax.ShapeDtypeStruct((1, 1), _f32),
    )(qn, posf, zsum)


def kernel(q, q_l, feature_bank, idx, neg_idx):
    idx_row = idx.reshape(1, B)
    posf = _k0(feature_bank, idx)
    Lmat, qn = _k1(q, q_l, feature_bank)
    zsum = _k2(Lmat, neg_idx.reshape(B, K))
    upd = _k3(qn, posf, idx_row)
    new_bank = _k4(feature_bank, idx_row, upd)
    loss = _k5(qn, posf, zsum)
    return loss[0, 0], new_bank
